# baseline (speedup 1.0000x reference)
import functools

import jax
import jax.numpy as jnp
from jax import lax
from jax.experimental import pallas as pl
from jax.experimental.pallas import tpu as pltpu
from jax.experimental.pallas import tpu_sc as plsc

N = 10000
D = 128
E = 320000
THETA = (0.5, 0.25, 0.125, 0.0625)

NC, NS = 2, 16
CHUNK = 128
CPT = 80
EPT = CHUNK * CPT
EP = NC * NS * EPT
NPAD = 10112
RPT = NPAD // NS

BN = 1000


def _sc_degree_body(dst_hbm, zeros_hbm, ones_hbm, out_hbm, deg_s, didx_v, ones_v):
  c = lax.axis_index("c")
  s = lax.axis_index("s")
  r0 = s * RPT
  pltpu.sync_copy(zeros_hbm.at[pl.ds(r0, RPT)], deg_s.at[pl.ds(r0, RPT)])
  pltpu.sync_copy(dst_hbm.at[c, s], didx_v)
  pltpu.sync_copy(ones_hbm, ones_v)
  plsc.subcore_barrier()

  @pl.loop(0, CPT)
  def _(j):
    pltpu.sync_copy(ones_v, deg_s.at[didx_v.at[j]], add=True)

  plsc.subcore_barrier()
  pltpu.sync_copy(deg_s.at[pl.ds(r0, RPT)], out_hbm.at[c, pl.ds(r0, RPT)])


@functools.lru_cache(maxsize=None)
def _sc_degree():
  return pl.kernel(
      _sc_degree_body,
      out_type=jax.ShapeDtypeStruct((NC, NPAD, D), jnp.float32),
      mesh=plsc.VectorSubcoreMesh(core_axis_name="c", subcore_axis_name="s",
                                  num_cores=NC, num_subcores=NS),
      scratch_types=[
          pltpu.VMEM_SHARED((NPAD, D), jnp.float32),
          pltpu.VMEM((CPT, CHUNK), jnp.int32),
          pltpu.VMEM((CHUNK, D), jnp.float32),
      ],
  )


SUB = CHUNK // 2
NSUB = 2 * CPT


def _sc_agg_body(y_hbm, src_hbm, dst_hbm, zeros_hbm, out_hbm,
                 agg_s, sidx_v, didx_v, ga_v, gb_v, sema, semb):
  c = lax.axis_index("c")
  s = lax.axis_index("s")
  r0 = s * RPT
  pltpu.sync_copy(zeros_hbm.at[pl.ds(r0, RPT)], agg_s.at[pl.ds(r0, RPT)])
  pltpu.sync_copy(src_hbm.at[c, s], sidx_v)
  pltpu.sync_copy(dst_hbm.at[c, s], didx_v)
  plsc.subcore_barrier()

  QS = SUB // 2

  def src_q(q):
    return y_hbm.at[sidx_v.at[q // 4, pl.ds((q % 4) * QS, QS)]]

  def buf_q(gv, q):
    return gv.at[pl.ds((q % 2) * QS, QS)]

  for q in range(4):
    pltpu.async_copy(src_q(q), buf_q((ga_v, gb_v)[q // 2], q),
                     (sema, semb)[q // 2])

  @pl.loop(0, NSUB, step=2)
  def _(g):
    for b, gv, sem in ((0, ga_v, sema), (1, gb_v, semb)):
      r = g + b
      pltpu.make_async_copy(src_q(2 * r), buf_q(gv, 0), sem).wait()
      pltpu.make_async_copy(src_q(2 * r + 1), buf_q(gv, 1), sem).wait()
      pltpu.sync_copy(gv, agg_s.at[didx_v.at[r]], add=True)

      @pl.when(r + 2 < NSUB)
      def _():
        pltpu.async_copy(src_q(2 * r + 4), buf_q(gv, 0), sem)
        pltpu.async_copy(src_q(2 * r + 5), buf_q(gv, 1), sem)

  plsc.subcore_barrier()
  pltpu.sync_copy(agg_s.at[pl.ds(r0, RPT)], out_hbm.at[c, pl.ds(r0, RPT)])


@functools.lru_cache(maxsize=None)
def _sc_agg():
  return pl.kernel(
      _sc_agg_body,
      out_type=jax.ShapeDtypeStruct((NC, NPAD, D), jnp.float32),
      mesh=plsc.VectorSubcoreMesh(core_axis_name="c", subcore_axis_name="s",
                                  num_cores=NC, num_subcores=NS),
      scratch_types=[
          pltpu.VMEM_SHARED((NPAD, D), jnp.float32),
          pltpu.VMEM((CPT, CHUNK), jnp.int32),
          pltpu.VMEM((NSUB, SUB), jnp.int32),
          pltpu.VMEM((SUB, D), jnp.float32),
          pltpu.VMEM((SUB, D), jnp.float32),
          pltpu.SemaphoreType.DMA,
          pltpu.SemaphoreType.DMA,
      ],
  )


def _deg_cols(deg_ref):
  d = deg_ref[0, :, 0] + deg_ref[1, :, 0]
  return jnp.maximum(d, 1.0)


def _pre_body(x_ref, deg_ref, w1_ref, b1_ref, w2d_ref, b2d_ref, emba_ref,
              embb_ref, y0_ref, s0_ref):
  x = x_ref[...]
  h1 = jnp.maximum(
      jnp.dot(x, w1_ref[...], preferred_element_type=jnp.float32,
              precision=lax.Precision.HIGHEST) + b1_ref[...], 0.0)
  l = jnp.dot(h1, w2d_ref[...], preferred_element_type=jnp.float32,
              precision=lax.Precision.HIGHEST) + b2d_ref[...]
  p = 1.0 / (1.0 + jnp.exp(-l))
  xk = x + emba_ref[...] + p * embb_ref[...]
  d = _deg_cols(deg_ref)
  y0_ref[...] = xk * lax.rsqrt(d)[:, None]
  s0_ref[...] = THETA[0] * xk


def _upd_body(theta, y_ref, agg_ref, deg_ref, s_ref, yn_ref, sn_ref):
  d = _deg_cols(deg_ref)
  a = agg_ref[0] + agg_ref[1]
  yn = y_ref[...] - a * (1.0 / d)[:, None]
  yn_ref[...] = yn
  sn_ref[...] = s_ref[...] + theta * yn * jnp.sqrt(d)[:, None]


def _fin_body(y_ref, agg_ref, deg_ref, s_ref, x_ref, e_ref, x0_ref,
              wt_ref, wb_ref, lt_ref, lb_ref, lbias_ref, o_ref):
  d = _deg_cols(deg_ref)
  a = agg_ref[0] + agg_ref[1]
  y3 = y_ref[...] - a * (1.0 / d)[:, None]
  hi = s_ref[...] + THETA[3] * y3 * jnp.sqrt(d)[:, None]
  dot = functools.partial(jnp.dot, preferred_element_type=jnp.float32,
                          precision=lax.Precision.HIGHEST)
  o = dot(hi, wt_ref[...]) + dot(x_ref[...], wb_ref[...])
  t = dot(e_ref[...], lt_ref[...]) + dot(o, lb_ref[...]) + lbias_ref[...]
  o_ref[...] = jnp.where(t > 0.0, t, 0.01 * t) + x0_ref[...]


_row_spec = pl.BlockSpec((BN, D), lambda i: (i, 0))
_agg_spec = pl.BlockSpec((NC, BN, D), lambda i: (0, i, 0))
_deg_spec = pl.BlockSpec((NC, BN, D), lambda i: (0, i, 0))
_mat_spec = pl.BlockSpec((D, D), lambda i: (0, 0))
_vec_spec = pl.BlockSpec((1, D), lambda i: (0, 0))
_f32 = functools.partial(jax.ShapeDtypeStruct, dtype=jnp.float32)


def _pre_call(x, deg2, w1, b1, w2d, b2d, emba, embb):
  return pl.pallas_call(
      _pre_body,
      grid=(N // BN,),
      in_specs=[_row_spec, _deg_spec, _mat_spec,
                _vec_spec, pl.BlockSpec((D, 1), lambda i: (0, 0)),
                pl.BlockSpec((1, 1), lambda i: (0, 0)),
                _vec_spec, _vec_spec],
      out_specs=[_row_spec, _row_spec],
      out_shape=[_f32((N, D)), _f32((N, D))],
  )(x, deg2, w1, b1, w2d, b2d, emba, embb)


def _upd_call(theta, y, agg, deg2, s):
  return pl.pallas_call(
      functools.partial(_upd_body, theta),
      grid=(N // BN,),
      in_specs=[_row_spec, _agg_spec, _deg_spec, _row_spec],
      out_specs=[_row_spec, _row_spec],
      out_shape=[_f32((N, D)), _f32((N, D))],
  )(y, agg, deg2, s)


def _fin_call(y, agg, deg2, s, x, e, x0, wt, wb, lt, lb, lbias):
  return pl.pallas_call(
      _fin_body,
      grid=(N // BN,),
      in_specs=[_row_spec, _agg_spec, _deg_spec, _row_spec, _row_spec,
                _row_spec, _row_spec, _mat_spec, _mat_spec, _mat_spec,
                _mat_spec, _vec_spec],
      out_specs=_row_spec,
      out_shape=_f32((N, D)),
  )(y, agg, deg2, s, x, e, x0, wt, wb, lt, lb, lbias)


def kernel(x, x0, e, edge_index, labels, nid, weights, pred_W1, pred_b1,
           pred_W2, pred_b2, emb_table, lin_W, lin_b, alpha):
  src = edge_index[0]
  dst = edge_index[1]
  nch = NC * NS * CPT
  sl = E // nch
  packed = jnp.sort((dst << 14) | src)
  key_t = packed.reshape(sl, nch).T
  src_t = key_t & 16383
  dst_t = key_t >> 14
  pad_src = jnp.zeros((nch, CHUNK - sl), jnp.int32)
  pad_dst = jnp.broadcast_to(
      N + jnp.arange(CHUNK - sl, dtype=jnp.int32), (nch, CHUNK - sl))
  srcp = jnp.concatenate([src_t, pad_src], axis=1).reshape(NC, NS, CPT, CHUNK)
  dstp = jnp.concatenate([dst_t, pad_dst], axis=1).reshape(NC, NS, CPT, CHUNK)
  dstp2 = dstp.reshape(NC, NS, NSUB, SUB)
  zeros128 = jnp.zeros((NPAD, D), jnp.float32)
  ones128 = jnp.ones((CHUNK, D), jnp.float32)

  w2d = (pred_W2[:, 1] - pred_W2[:, 0]).reshape(D, 1)
  b2d = (pred_b2[1] - pred_b2[0]).reshape(1, 1)
  emba = (alpha * emb_table[0]).reshape(1, D)
  embb = (alpha * (emb_table[1] - emb_table[0])).reshape(1, D)
  b1 = pred_b1.reshape(1, D)
  wt, wb = weights[:D], weights[D:]
  lt, lb = lin_W[:D], lin_W[D:]
  lbias = lin_b.reshape(1, D)

  deg2 = _sc_degree()(dstp, zeros128, ones128)
  y, s = _pre_call(x, deg2, pred_W1, b1, w2d, b2d, emba, embb)
  for k in (1, 2):
    agg = _sc_agg()(y, srcp, dstp2, zeros128)
    y, s = _upd_call(THETA[k], y, agg, deg2, s)
  agg = _sc_agg()(y, srcp, dstp2, zeros128)
  return _fin_call(y, agg, deg2, s, x, e, x0, wt, wb, lt, lb, lbias)


# speedup vs baseline: 1.0011x; 1.0011x over previous
import functools

import jax
import jax.numpy as jnp
from jax import lax
from jax.experimental import pallas as pl
from jax.experimental.pallas import tpu as pltpu
from jax.experimental.pallas import tpu_sc as plsc

N = 10000
D = 128
E = 320000
THETA = (0.5, 0.25, 0.125, 0.0625)

NC, NS = 2, 16
CHUNK = 128
CPT = 80
EPT = CHUNK * CPT
EP = NC * NS * EPT
NPAD = 10112
RPT = NPAD // NS

BN = 1000


def _sc_degree_body(dst_hbm, zeros_hbm, ones_hbm, out_hbm, deg_s, didx_v, ones_v):
  c = lax.axis_index("c")
  s = lax.axis_index("s")
  r0 = s * RPT
  pltpu.sync_copy(zeros_hbm.at[pl.ds(r0, RPT)], deg_s.at[pl.ds(r0, RPT)])
  pltpu.sync_copy(dst_hbm.at[c, s], didx_v)
  pltpu.sync_copy(ones_hbm, ones_v)
  plsc.subcore_barrier()

  @pl.loop(0, CPT)
  def _(j):
    pltpu.sync_copy(ones_v, deg_s.at[didx_v.at[j]], add=True)

  plsc.subcore_barrier()
  pltpu.sync_copy(deg_s.at[pl.ds(r0, RPT)], out_hbm.at[c, pl.ds(r0, RPT)])


@functools.lru_cache(maxsize=None)
def _sc_degree():
  return pl.kernel(
      _sc_degree_body,
      out_type=jax.ShapeDtypeStruct((NC, NPAD, D), jnp.float32),
      mesh=plsc.VectorSubcoreMesh(core_axis_name="c", subcore_axis_name="s",
                                  num_cores=NC, num_subcores=NS),
      scratch_types=[
          pltpu.VMEM_SHARED((NPAD, D), jnp.float32),
          pltpu.VMEM((CPT, CHUNK), jnp.int32),
          pltpu.VMEM((CHUNK, D), jnp.float32),
      ],
  )


SUB = CHUNK // 2
NSUB = 2 * CPT


def _sc_agg_body(y_hbm, src_hbm, dst_hbm, zeros_hbm, out_hbm,
                 agg_s, sidx_v, didx_v, ga_v, gb_v, sema, semb):
  c = lax.axis_index("c")
  s = lax.axis_index("s")
  r0 = s * RPT
  pltpu.sync_copy(zeros_hbm.at[pl.ds(r0, RPT)], agg_s.at[pl.ds(r0, RPT)])
  pltpu.sync_copy(src_hbm.at[c, s], sidx_v)
  pltpu.sync_copy(dst_hbm.at[c, s], didx_v)
  plsc.subcore_barrier()

  def src_at(r):
    return y_hbm.at[sidx_v.at[r // 2, pl.ds((r % 2) * SUB, SUB)]]

  pltpu.async_copy(src_at(0), ga_v, sema)
  pltpu.async_copy(src_at(1), gb_v, semb)

  @pl.loop(0, NSUB, step=2)
  def _(g):
    for b, gv, sem in ((0, ga_v, sema), (1, gb_v, semb)):
      r = g + b
      pltpu.make_async_copy(src_at(r), gv, sem).wait()
      pltpu.sync_copy(gv, agg_s.at[didx_v.at[r]], add=True)

      @pl.when(r + 2 < NSUB)
      def _():
        pltpu.async_copy(src_at(r + 2), gv, sem)

  plsc.subcore_barrier()
  pltpu.sync_copy(agg_s.at[pl.ds(r0, RPT)], out_hbm.at[c, pl.ds(r0, RPT)])


@functools.lru_cache(maxsize=None)
def _sc_agg():
  return pl.kernel(
      _sc_agg_body,
      out_type=jax.ShapeDtypeStruct((NC, NPAD, D), jnp.float32),
      mesh=plsc.VectorSubcoreMesh(core_axis_name="c", subcore_axis_name="s",
                                  num_cores=NC, num_subcores=NS),
      scratch_types=[
          pltpu.VMEM_SHARED((NPAD, D), jnp.float32),
          pltpu.VMEM((CPT, CHUNK), jnp.int32),
          pltpu.VMEM((NSUB, SUB), jnp.int32),
          pltpu.VMEM((SUB, D), jnp.float32),
          pltpu.VMEM((SUB, D), jnp.float32),
          pltpu.SemaphoreType.DMA,
          pltpu.SemaphoreType.DMA,
      ],
  )


def _deg_cols(deg_ref):
  d = deg_ref[0, :, 0] + deg_ref[1, :, 0]
  return jnp.maximum(d, 1.0)


def _pre_body(x_ref, deg_ref, w1_ref, b1_ref, w2d_ref, b2d_ref, emba_ref,
              embb_ref, y0_ref, s0_ref):
  x = x_ref[...]
  h1 = jnp.maximum(
      jnp.dot(x, w1_ref[...], preferred_element_type=jnp.float32,
              precision=lax.Precision.HIGHEST) + b1_ref[...], 0.0)
  l = jnp.dot(h1, w2d_ref[...], preferred_element_type=jnp.float32,
              precision=lax.Precision.HIGHEST) + b2d_ref[...]
  p = 1.0 / (1.0 + jnp.exp(-l))
  xk = x + emba_ref[...] + p * embb_ref[...]
  d = _deg_cols(deg_ref)
  y0_ref[...] = xk * lax.rsqrt(d)[:, None]
  s0_ref[...] = THETA[0] * xk


def _upd_body(theta, y_ref, agg_ref, deg_ref, s_ref, yn_ref, sn_ref):
  d = _deg_cols(deg_ref)
  a = agg_ref[0] + agg_ref[1]
  yn = y_ref[...] - a * (1.0 / d)[:, None]
  yn_ref[...] = yn
  sn_ref[...] = s_ref[...] + theta * yn * jnp.sqrt(d)[:, None]


def _fin_body(y_ref, agg_ref, deg_ref, s_ref, x_ref, e_ref, x0_ref,
              wt_ref, wb_ref, lt_ref, lb_ref, lbias_ref, o_ref):
  d = _deg_cols(deg_ref)
  a = agg_ref[0] + agg_ref[1]
  y3 = y_ref[...] - a * (1.0 / d)[:, None]
  hi = s_ref[...] + THETA[3] * y3 * jnp.sqrt(d)[:, None]
  dot = functools.partial(jnp.dot, preferred_element_type=jnp.float32,
                          precision=lax.Precision.HIGHEST)
  o = dot(hi, wt_ref[...]) + dot(x_ref[...], wb_ref[...])
  t = dot(e_ref[...], lt_ref[...]) + dot(o, lb_ref[...]) + lbias_ref[...]
  o_ref[...] = jnp.where(t > 0.0, t, 0.01 * t) + x0_ref[...]


_row_spec = pl.BlockSpec((BN, D), lambda i: (i, 0))
_agg_spec = pl.BlockSpec((NC, BN, D), lambda i: (0, i, 0))
_deg_spec = pl.BlockSpec((NC, BN, D), lambda i: (0, i, 0))
_mat_spec = pl.BlockSpec((D, D), lambda i: (0, 0))
_vec_spec = pl.BlockSpec((1, D), lambda i: (0, 0))
_f32 = functools.partial(jax.ShapeDtypeStruct, dtype=jnp.float32)


def _pre_call(x, deg2, w1, b1, w2d, b2d, emba, embb):
  return pl.pallas_call(
      _pre_body,
      grid=(N // BN,),
      in_specs=[_row_spec, _deg_spec, _mat_spec,
                _vec_spec, pl.BlockSpec((D, 1), lambda i: (0, 0)),
                pl.BlockSpec((1, 1), lambda i: (0, 0)),
                _vec_spec, _vec_spec],
      out_specs=[_row_spec, _row_spec],
      out_shape=[_f32((N, D)), _f32((N, D))],
  )(x, deg2, w1, b1, w2d, b2d, emba, embb)


def _upd_call(theta, y, agg, deg2, s):
  return pl.pallas_call(
      functools.partial(_upd_body, theta),
      grid=(N // BN,),
      in_specs=[_row_spec, _agg_spec, _deg_spec, _row_spec],
      out_specs=[_row_spec, _row_spec],
      out_shape=[_f32((N, D)), _f32((N, D))],
  )(y, agg, deg2, s)


def _fin_call(y, agg, deg2, s, x, e, x0, wt, wb, lt, lb, lbias):
  return pl.pallas_call(
      _fin_body,
      grid=(N // BN,),
      in_specs=[_row_spec, _agg_spec, _deg_spec, _row_spec, _row_spec,
                _row_spec, _row_spec, _mat_spec, _mat_spec, _mat_spec,
                _mat_spec, _vec_spec],
      out_specs=_row_spec,
      out_shape=_f32((N, D)),
  )(y, agg, deg2, s, x, e, x0, wt, wb, lt, lb, lbias)


def kernel(x, x0, e, edge_index, labels, nid, weights, pred_W1, pred_b1,
           pred_W2, pred_b2, emb_table, lin_W, lin_b, alpha):
  src = edge_index[0]
  dst = edge_index[1]
  nch = NC * NS * CPT
  sl = E // nch
  packed = jnp.sort((dst << 14) | src)
  key_t = packed.reshape(sl, nch).T
  src_t = key_t & 16383
  dst_t = key_t >> 14
  pad_src = jnp.zeros((nch, CHUNK - sl), jnp.int32)
  pad_dst = jnp.broadcast_to(
      N + jnp.arange(CHUNK - sl, dtype=jnp.int32), (nch, CHUNK - sl))
  srcp = jnp.concatenate([src_t, pad_src], axis=1).reshape(NC, NS, CPT, CHUNK)
  dstp = jnp.concatenate([dst_t, pad_dst], axis=1).reshape(NC, NS, CPT, CHUNK)
  dstp2 = dstp.reshape(NC, NS, NSUB, SUB)
  zeros128 = jnp.zeros((NPAD, D), jnp.float32)
  ones128 = jnp.ones((CHUNK, D), jnp.float32)

  w2d = (pred_W2[:, 1] - pred_W2[:, 0]).reshape(D, 1)
  b2d = (pred_b2[1] - pred_b2[0]).reshape(1, 1)
  emba = (alpha * emb_table[0]).reshape(1, D)
  embb = (alpha * (emb_table[1] - emb_table[0])).reshape(1, D)
  b1 = pred_b1.reshape(1, D)
  wt, wb = weights[:D], weights[D:]
  lt, lb = lin_W[:D], lin_W[D:]
  lbias = lin_b.reshape(1, D)

  deg2 = _sc_degree()(dstp, zeros128, ones128)
  y, s = _pre_call(x, deg2, pred_W1, b1, w2d, b2d, emba, embb)
  for k in (1, 2):
    agg = _sc_agg()(y, srcp, dstp2, zeros128)
    y, s = _upd_call(THETA[k], y, agg, deg2, s)
  agg = _sc_agg()(y, srcp, dstp2, zeros128)
  return _fin_call(y, agg, deg2, s, x, e, x0, wt, wb, lt, lb, lbias)
